# Initial kernel scaffold; baseline (speedup 1.0000x reference)
import jax
import jax.numpy as jnp
from jax.experimental import pallas as pl


def kernel(comp_x, sg_x, lat_x, sg_table, cW1, cb1, cg1, cbeta1, cW2, cb2, cg2, cbeta2, lW1, lb1, lg1, lbeta1, lW2, lb2, lg2, lbeta2, pW1, pb1, pg1, pbeta1, pW2, pb2, pg2, pbeta2, pW3, pb3, pg3, pbeta3, pW4, pb4):
    raise NotImplementedError("write your pallas kernel here")


# speedup vs baseline: 1.1863x; 1.1863x over previous
import functools

import jax
import jax.numpy as jnp
import numpy as np
from jax import lax
from jax.experimental import pallas as pl
from jax.experimental.pallas import tpu as pltpu
from jax.experimental.pallas import tpu_sc as plsc

EPS = 1e-5
B = 16384
BM = 512


def _sc_gather(table, idx):
    info = plsc.get_sparse_core_info()
    nc, ns = info.num_cores, info.num_subcores
    nw = nc * ns
    b_per_w = B // nw
    n_chunks = b_per_w // 128
    idx2d = idx.reshape(B // 128, 128)
    d = table.shape[1]

    @functools.partial(
        pl.kernel,
        out_type=jax.ShapeDtypeStruct((B, d), jnp.float32),
        mesh=plsc.VectorSubcoreMesh(core_axis_name="c", subcore_axis_name="s"),
        scratch_types=[
            pltpu.VMEM((n_chunks, 128), jnp.int32),
            pltpu.VMEM((b_per_w, d), jnp.float32),
            pltpu.SemaphoreType.DMA,
        ],
    )
    def k(table_hbm, idx_hbm, out_hbm, idx_v, rows_v, sem):
        wid = lax.axis_index("s") * nc + lax.axis_index("c")
        base = wid * b_per_w
        pltpu.sync_copy(idx_hbm.at[pl.ds(wid * n_chunks, n_chunks)], idx_v)
        copies = []
        for j in range(n_chunks):
            copies.append(
                pltpu.async_copy(
                    table_hbm.at[idx_v.at[j]],
                    rows_v.at[pl.ds(j * 128, 128)],
                    sem,
                )
            )
        for c in copies:
            c.wait()
        pltpu.sync_copy(rows_v, out_hbm.at[pl.ds(base, b_per_w)])

    return k(table, idx2d)


def _mlp_body(comp_ref, sg_ref, lat_ref, a1_ref, a3_ref, a2_ref, b1_ref,
              w2_ref, b2_ref, w3_ref, b3_ref, w4_ref, b4_ref, out_ref):
    h = (
        jnp.dot(comp_ref[...], a1_ref[...], preferred_element_type=jnp.float32)
        + jnp.dot(sg_ref[...], a3_ref[...], preferred_element_type=jnp.float32)
        + jnp.dot(lat_ref[...], a2_ref[...], preferred_element_type=jnp.float32)
        + b1_ref[...]
    )
    h = jnp.maximum(h, 0.2 * h)
    h = jnp.dot(h, w2_ref[...], preferred_element_type=jnp.float32) + b2_ref[...]
    h = jnp.maximum(h, 0.2 * h)
    h = jnp.dot(h, w3_ref[...], preferred_element_type=jnp.float32) + b3_ref[...]
    h = jnp.maximum(h, 0.2 * h)
    out_ref[...] = (
        jnp.dot(h, w4_ref[...], preferred_element_type=jnp.float32) + b4_ref[...]
    )


def _row_spec(bm, d):
    return pl.BlockSpec((bm, d), lambda i: (i, 0))


def _full_spec(shape):
    return pl.BlockSpec(shape, lambda i: (0, 0))


def _tc_mlp(comp_x, sg_emb, lat8, a1, a3, a2, b1, w2, b2, w3, b3, w4, b4):
    grid = (B // BM,)
    return pl.pallas_call(
        _mlp_body,
        grid=grid,
        in_specs=[
            _row_spec(BM, 128),
            _row_spec(BM, 128),
            _row_spec(BM, 8),
            _full_spec(a1.shape),
            _full_spec(a3.shape),
            _full_spec(a2.shape),
            _full_spec(b1.shape),
            _full_spec(w2.shape),
            _full_spec(b2.shape),
            _full_spec(w3.shape),
            _full_spec(b3.shape),
            _full_spec(w4.shape),
            _full_spec(b4.shape),
        ],
        out_specs=_row_spec(BM, 1),
        out_shape=jax.ShapeDtypeStruct((B, 1), jnp.float32),
    )(comp_x, sg_emb, lat8, a1, a3, a2, b1, w2, b2, w3, b3, w4, b4)


def kernel(comp_x, sg_x, lat_x, sg_table, cW1, cb1, cg1, cbeta1, cW2, cb2, cg2,
           cbeta2, lW1, lb1, lg1, lbeta1, lW2, lb2, lg2, lbeta2, pW1, pb1, pg1,
           pbeta1, pW2, pb2, pg2, pbeta2, pW3, pb3, pg3, pbeta3, pW4, pb4):
    s = np.float32(1.0 / np.sqrt(1.0 + EPS))

    a1c, a2c = cg1 * s, cg2 * s
    mc = (cW1.T * a1c[None, :]) @ (cW2.T * a2c[None, :])
    vc = (a1c * cb1 + cbeta1) @ (cW2.T * a2c[None, :]) + (a2c * cb2 + cbeta2)
    a1l, a2l = lg1 * s, lg2 * s
    ml = (lW1.T * a1l[None, :]) @ (lW2.T * a2l[None, :])
    vl = (a1l * lb1 + lbeta1) @ (lW2.T * a2l[None, :]) + (a2l * lb2 + lbeta2)

    p1 = pW1.T
    p1c, p1s, p1l = p1[:256], p1[256:320], p1[320:]
    ap1 = pg1 * s
    a1m = (mc @ p1c) * ap1[None, :]
    a3m = jnp.concatenate(
        [p1s * ap1[None, :], jnp.zeros((64, 512), jnp.float32)], axis=0
    )
    a2m = (ml @ p1l) * ap1[None, :]
    b1v = ap1 * (vc @ p1c + vl @ p1l + pb1) + pbeta1
    a2m = jnp.concatenate([a2m, jnp.zeros((2, 512), jnp.float32)], axis=0)

    ap2 = pg2 * s
    w2 = pW2.T * ap2[None, :]
    b2v = ap2 * pb2 + pbeta2
    ap3 = pg3 * s
    w3 = pW3.T * ap3[None, :]
    b3v = ap3 * pb3 + pbeta3
    w4 = pW4.T
    b4v = pb4

    lat8 = jnp.concatenate([lat_x, jnp.zeros((B, 2), jnp.float32)], axis=1)
    idx = sg_x[:, 0].astype(jnp.int32)
    table_pad = jnp.pad(sg_table, ((0, 2), (0, 64)))
    sg_emb = _sc_gather(table_pad, idx)

    return _tc_mlp(
        comp_x, sg_emb, lat8,
        a1m, a3m, a2m, b1v.reshape(1, 512),
        w2, b2v.reshape(1, 256),
        w3, b3v.reshape(1, 128),
        w4, b4v.reshape(1, 1),
    )
